# baseline (speedup 1.0000x reference)
import dataclasses
import functools

import jax
import jax.numpy as jnp
from jax import lax
from jax.experimental import pallas as pl
from jax.experimental.pallas import tpu as pltpu
from jax.experimental.pallas import tpu_sc as plsc

F32 = jnp.float32
I32 = jnp.int32

_EB = 128
_NSUB = 16
_NCORE = 2
_ACC_ROWS = 25088


def _mesh():
    return plsc.VectorSubcoreMesh(core_axis_name="c", subcore_axis_name="s")


def _sc_params(tc_tiling=True):
    cp = pltpu.CompilerParams()
    fields = pltpu.CompilerParams.__dataclass_fields__
    if "needs_layout_passes" in fields:
        cp = dataclasses.replace(cp, needs_layout_passes=False)
    if not tc_tiling and "use_tc_tiling_on_sc" in fields:
        cp = dataclasses.replace(cp, use_tc_tiling_on_sc=False)
    return cp


@functools.partial(jax.jit, static_argnames=("n_dst",))
def _segsum(src2d, dst2d, t0, t1, zz, *, n_dst):
    rows2d = src2d.shape[0]
    per_tile = rows2d // _NSUB
    half = n_dst // 2
    stripe = (half // _NSUB) // 8 * 8
    rem = half - _NSUB * stripe
    acc_rows = _ACC_ROWS
    assert acc_rows >= half + 1

    chunk = 32
    n_chunks = per_tile // chunk

    @functools.partial(
        pl.kernel,
        out_type=(jax.ShapeDtypeStruct((n_dst, 64), F32),
                  jax.ShapeDtypeStruct((n_dst, 64), F32)),
        mesh=_mesh(),
        compiler_params=_sc_params(tc_tiling=False),
        scratch_types=[
            pltpu.VMEM((chunk, _EB), I32),
            pltpu.VMEM((chunk, _EB), I32),
            pltpu.VMEM((_EB, 64), F32),
            pltpu.VMEM((_EB, 64), F32),
            pltpu.VMEM_SHARED((acc_rows, 64), F32),
            pltpu.SemaphoreType.DMA,
            pltpu.SemaphoreType.DMA,
            pltpu.SemaphoreType.DMA,
            pltpu.SemaphoreType.DMA,
        ],
    )
    def seg(src_hbm, dst_hbm, t0_hbm, t1_hbm, zz_hbm, s0_hbm, s1_hbm,
            src_st, off_st, rows0, rows1, acc, gs0, gs1, ss0, ss1):
        c = lax.axis_index("c")
        s = lax.axis_index("s")
        base = c * half
        zstripe = acc_rows // _NSUB
        rows = (rows0, rows1)
        gsem = (gs0, gs1)
        ssem = (ss0, ss1)

        for t_hbm, s_hbm in ((t0_hbm, s0_hbm), (t1_hbm, s1_hbm)):
            pltpu.sync_copy(zz_hbm, acc.at[pl.ds(s * zstripe, zstripe)])
            plsc.subcore_barrier()

            @pl.loop(0, n_chunks)
            def _(ch):
                row0 = s * per_tile + ch * chunk
                pltpu.sync_copy(src_hbm.at[pl.ds(row0, chunk)], src_st)
                pltpu.sync_copy(dst_hbm.at[pl.ds(row0, chunk)], off_st)

                @pl.loop(0, chunk)
                def _(b):
                    for j in range(_EB // 16):
                        v = off_st[b, pl.ds(j * 16, 16)]
                        o = v - base
                        ok = (o >= 0) & (o < half)
                        off_st[b, pl.ds(j * 16, 16)] = jnp.where(
                            ok, o, half + (v & 63))

                gd = [None] * chunk
                sd = [None] * chunk
                gd[0] = pltpu.async_copy(
                    t_hbm.at[src_st.at[0]], rows[0], gsem[0])
                for b in range(chunk):
                    i = b & 1
                    if b + 1 < chunk:
                        if b >= 1:
                            sd[b - 1].wait()
                        gd[b + 1] = pltpu.async_copy(
                            t_hbm.at[src_st.at[b + 1]], rows[1 - i],
                            gsem[1 - i])
                    gd[b].wait()
                    sd[b] = pltpu.async_copy(
                        rows[i], acc.at[off_st.at[b]], ssem[i], add=True)
                sd[chunk - 2].wait()
                sd[chunk - 1].wait()

            plsc.subcore_barrier()
            pltpu.sync_copy(acc.at[pl.ds(s * stripe, stripe)],
                            s_hbm.at[pl.ds(base + s * stripe, stripe)])

            @pl.when(s == _NSUB - 1)
            def _():
                pltpu.sync_copy(acc.at[pl.ds(_NSUB * stripe, rem)],
                                s_hbm.at[pl.ds(base + _NSUB * stripe, rem)])

            plsc.subcore_barrier()

    return seg(src2d, dst2d, t0, t1, zz)


_CHE = 1024
_CAPC = 35
_BUFC = 2064


@functools.partial(jax.jit, static_argnames=("n_dst",))
def _binedges(sv2d, dv2d, sr2d, dr2d, *, n_dst):
    rows2d = sv2d.shape[0]
    per_tile = rows2d // _NSUB
    half = n_dst // 2

    flat_t = jax.ShapeDtypeStruct((2, _NSUB, 1, _CAPC * _CHE), I32)
    blk_t = jax.ShapeDtypeStruct((2, _NSUB, _CAPC * 8, _EB), I32)
    cnt_t = jax.ShapeDtypeStruct((2, _NSUB, 1, 16), I32)

    @functools.partial(
        pl.kernel,
        out_type=(flat_t, blk_t, cnt_t, flat_t, blk_t, cnt_t),
        mesh=_mesh(),
        compiler_params=_sc_params(tc_tiling=False),
        scratch_types=[
            pltpu.VMEM((32, _EB), I32),
            pltpu.VMEM((32, _EB), I32),
            pltpu.VMEM((_BUFC,), I32),
            pltpu.VMEM((_BUFC,), I32),
            pltpu.VMEM((_BUFC,), I32),
            pltpu.VMEM((_BUFC,), I32),
            pltpu.VMEM((8, _EB), I32),
            pltpu.VMEM((16,), I32),
        ],
    )
    def binker(sv_hbm, dv_hbm, sr_hbm, dr_hbm,
               fv_hbm, ov_hbm, kv_hbm, fr_hbm, orr_hbm, kr_hbm,
               st_s, st_d, bs0, bo0, bs1, bo1, fl2, kst):
        c = lax.axis_index("c")
        s = lax.axis_index("s")
        iota = lax.iota(I32, 16)

        def bin_et(src_hbm, dst_hbm, f_out, o_out, k_out):
            bufs = ((bs0, bo0), (bs1, bo1))

            def flush_chunk(bkt, off_in_buf, k):
                bsrc, boff = bufs[bkt]
                for r in range(8):
                    for j in range(8):
                        fl2[r, pl.ds(j * 16, 16)] = (
                            boff[pl.ds(off_in_buf + r * _EB + j * 16, 16)])
                pltpu.sync_copy(
                    bsrc.at[pl.ds(off_in_buf, _CHE)],
                    f_out.at[bkt, s, 0, pl.ds(k * _CHE, _CHE)])
                pltpu.sync_copy(fl2, o_out.at[bkt, s, pl.ds(k * 8, 8)])

            def maybe_flush(bkt):
                def do(args):
                    f, k = args
                    flush_chunk(bkt, 0, k)
                    bsrc, boff = bufs[bkt]
                    for j in range(9):
                        t = bsrc[pl.ds(_CHE + j * 16, 16)]
                        bsrc[pl.ds(j * 16, 16)] = t
                        t2 = boff[pl.ds(_CHE + j * 16, 16)]
                        boff[pl.ds(j * 16, 16)] = t2
                    return (f - _CHE, k + 1)

                def keep(args):
                    return args

                return lambda f, k: lax.cond(f >= _CHE, do, keep, (f, k))

            def chunk_body(ch, carry):
                row0 = s * per_tile + ch * 32
                pltpu.sync_copy(src_hbm.at[pl.ds(row0, 32)], st_s)
                pltpu.sync_copy(dst_hbm.at[pl.ds(row0, 32)], st_d)

                def row_body(r, carry):
                    f0, k0, f1, k1 = carry
                    for j in range(8):
                        sv = st_s[r, pl.ds(j * 16, 16)]
                        dv = st_d[r, pl.ds(j * 16, 16)]
                        m0 = dv < half
                        n0 = jnp.sum(m0.astype(I32), axis=0)
                        plsc.store_compressed(bs0.at[pl.ds(f0, 16)], sv, mask=m0)
                        plsc.store_compressed(bo0.at[pl.ds(f0, 16)], dv, mask=m0)
                        m1 = jnp.logical_not(m0)
                        plsc.store_compressed(bs1.at[pl.ds(f1, 16)], sv, mask=m1)
                        plsc.store_compressed(
                            bo1.at[pl.ds(f1, 16)], dv - half, mask=m1)
                        f0 = f0 + n0
                        f1 = f1 + (16 - n0)
                    f0, k0 = maybe_flush(0)(f0, k0)
                    f1, k1 = maybe_flush(1)(f1, k1)
                    return (f0, k0, f1, k1)

                return lax.fori_loop(0, 32, row_body, carry)

            f0, k0, f1, k1 = lax.fori_loop(
                0, per_tile // 32, chunk_body, (0, 0, 0, 0))

            def drain(bkt, f, k):
                bsrc, boff = bufs[bkt]
                bsrc[pl.ds(f, 16)] = iota + ((f * 37) & 16383)
                boff[pl.ds(f, 16)] = half + ((iota + f) & 63)
                f = (f & ~15) + 16

                def wcond(st):
                    return (st[0] & (_CHE - 1)) != 0

                def wbody(st):
                    fw = st[0]
                    bsrc[pl.ds(fw, 16)] = iota + ((fw * 37) & 16383)
                    boff[pl.ds(fw, 16)] = half + ((iota + fw) & 63)
                    return (fw + 16,)

                f = lax.while_loop(wcond, wbody, (f,))[0]

                def fl(i, kk):
                    flush_chunk(bkt, i * _CHE, kk)
                    return kk + 1

                return lax.fori_loop(0, f // _CHE, fl, k)

            k0 = drain(0, f0, k0)
            k1 = drain(1, f1, k1)
            kst[...] = jnp.broadcast_to(k0, (16,)).astype(I32)
            pltpu.sync_copy(kst, k_out.at[0, s, 0])
            kst[...] = jnp.broadcast_to(k1, (16,)).astype(I32)
            pltpu.sync_copy(kst, k_out.at[1, s, 0])

        @pl.when(c == 0)
        def _():
            bin_et(sv_hbm, dv_hbm, fv_hbm, ov_hbm, kv_hbm)

        @pl.when(c == 1)
        def _():
            bin_et(sr_hbm, dr_hbm, fr_hbm, orr_hbm, kr_hbm)

    return binker(sv2d, dv2d, sr2d, dr2d)


@functools.partial(jax.jit, static_argnames=("n_dst",))
def _seglayer(fv, ov, kv, fr, orr, kr, ta0, ta1, tb0, tb1, zz, *, n_dst):
    half = n_dst // 2
    stripe = (half // _NSUB) // 8 * 8
    rem = half - _NSUB * stripe
    acc_rows = _ACC_ROWS
    assert acc_rows >= half + 64

    @functools.partial(
        pl.kernel,
        out_type=tuple(jax.ShapeDtypeStruct((n_dst, 64), F32)
                       for _ in range(4)),
        mesh=_mesh(),
        compiler_params=_sc_params(tc_tiling=False),
        scratch_types=[
            pltpu.VMEM((_CHE,), I32),
            pltpu.VMEM((8, _EB), I32),
            pltpu.VMEM((_EB, 64), F32),
            pltpu.VMEM((_EB, 64), F32),
            pltpu.VMEM((16,), I32),
            pltpu.VMEM_SHARED((acc_rows, 64), F32),
            pltpu.SemaphoreType.DMA,
            pltpu.SemaphoreType.DMA,
            pltpu.SemaphoreType.DMA,
            pltpu.SemaphoreType.DMA,
        ],
    )
    def seg(fv_hbm, ov_hbm, kv_hbm, fr_hbm, orr_hbm, kr_hbm,
            ta0_hbm, ta1_hbm, tb0_hbm, tb1_hbm, zz_hbm,
            sva0_hbm, sva1_hbm, srb0_hbm, srb1_hbm,
            src_st, off_st, rows0, rows1, kst, acc, gs0, gs1, ss0, ss1):
        c = lax.axis_index("c")
        s = lax.axis_index("s")
        base = c * half
        zstripe = acc_rows // _NSUB
        rows = (rows0, rows1)
        gsem = (gs0, gs1)
        ssem = (ss0, ss1)

        for fsrc_hbm, foff_hbm, kcnt_hbm, passes in (
                (fv_hbm, ov_hbm, kv_hbm,
                 ((ta0_hbm, sva0_hbm), (ta1_hbm, sva1_hbm))),
                (fr_hbm, orr_hbm, kr_hbm,
                 ((tb0_hbm, srb0_hbm), (tb1_hbm, srb1_hbm)))):
            pltpu.sync_copy(kcnt_hbm.at[c, s, 0], kst)
            nck = jnp.max(kst[...], axis=0)

            for t_hbm, s_hbm in passes:
                pltpu.sync_copy(zz_hbm, acc.at[pl.ds(s * zstripe, zstripe)])
                plsc.subcore_barrier()

                @pl.loop(0, nck)
                def _(k):
                    pltpu.sync_copy(
                        fsrc_hbm.at[c, s, 0, pl.ds(k * _CHE, _CHE)], src_st)
                    pltpu.sync_copy(
                        foff_hbm.at[c, s, pl.ds(k * 8, 8)], off_st)

                    gd = [None] * 8
                    sd = [None] * 8
                    gd[0] = pltpu.async_copy(
                        t_hbm.at[src_st.at[pl.ds(0, _EB)]], rows[0], gsem[0])
                    for b in range(8):
                        i = b & 1
                        if b + 1 < 8:
                            if b >= 1:
                                sd[b - 1].wait()
                            gd[b + 1] = pltpu.async_copy(
                                t_hbm.at[src_st.at[pl.ds((b + 1) * _EB, _EB)]],
                                rows[1 - i], gsem[1 - i])
                        gd[b].wait()
                        sd[b] = pltpu.async_copy(
                            rows[i], acc.at[off_st.at[b]], ssem[i], add=True)
                    sd[6].wait()
                    sd[7].wait()

                plsc.subcore_barrier()
                pltpu.sync_copy(acc.at[pl.ds(s * stripe, stripe)],
                                s_hbm.at[pl.ds(base + s * stripe, stripe)])

                @pl.when(s == _NSUB - 1)
                def _():
                    pltpu.sync_copy(
                        acc.at[pl.ds(_NSUB * stripe, rem)],
                        s_hbm.at[pl.ds(base + _NSUB * stripe, rem)])

                plsc.subcore_barrier()

    return seg(fv, ov, kv, fr, orr, kr, ta0, ta1, tb0, tb1, zz)


@functools.partial(jax.jit, static_argnames=("n_dst",))
def _segsum_b(fsrc, foff, kcnt, t0, t1, zz, *, n_dst):
    half = n_dst // 2
    stripe = (half // _NSUB) // 8 * 8
    rem = half - _NSUB * stripe
    acc_rows = _ACC_ROWS
    assert acc_rows >= half + 64

    @functools.partial(
        pl.kernel,
        out_type=(jax.ShapeDtypeStruct((n_dst, 64), F32),
                  jax.ShapeDtypeStruct((n_dst, 64), F32)),
        mesh=_mesh(),
        compiler_params=_sc_params(tc_tiling=False),
        scratch_types=[
            pltpu.VMEM((_CHE,), I32),
            pltpu.VMEM((8, _EB), I32),
            pltpu.VMEM((_EB, 64), F32),
            pltpu.VMEM((_EB, 64), F32),
            pltpu.VMEM((16,), I32),
            pltpu.VMEM_SHARED((acc_rows, 64), F32),
            pltpu.SemaphoreType.DMA,
            pltpu.SemaphoreType.DMA,
            pltpu.SemaphoreType.DMA,
            pltpu.SemaphoreType.DMA,
        ],
    )
    def seg(fsrc_hbm, foff_hbm, kcnt_hbm, t0_hbm, t1_hbm, zz_hbm,
            s0_hbm, s1_hbm,
            src_st, off_st, rows0, rows1, kst, acc, gs0, gs1, ss0, ss1):
        c = lax.axis_index("c")
        s = lax.axis_index("s")
        base = c * half
        zstripe = acc_rows // _NSUB
        rows = (rows0, rows1)
        gsem = (gs0, gs1)
        ssem = (ss0, ss1)

        pltpu.sync_copy(kcnt_hbm.at[c, s, 0], kst)
        nck = jnp.max(kst[...], axis=0)

        for t_hbm, s_hbm in ((t0_hbm, s0_hbm), (t1_hbm, s1_hbm)):
            pltpu.sync_copy(zz_hbm, acc.at[pl.ds(s * zstripe, zstripe)])
            plsc.subcore_barrier()

            @pl.loop(0, nck)
            def _(k):
                pltpu.sync_copy(
                    fsrc_hbm.at[c, s, 0, pl.ds(k * _CHE, _CHE)], src_st)
                pltpu.sync_copy(foff_hbm.at[c, s, pl.ds(k * 8, 8)], off_st)

                gd = [None] * 8
                sd = [None] * 8
                gd[0] = pltpu.async_copy(
                    t_hbm.at[src_st.at[pl.ds(0, _EB)]], rows[0], gsem[0])
                for b in range(8):
                    i = b & 1
                    if b + 1 < 8:
                        if b >= 1:
                            sd[b - 1].wait()
                        gd[b + 1] = pltpu.async_copy(
                            t_hbm.at[src_st.at[pl.ds((b + 1) * _EB, _EB)]],
                            rows[1 - i], gsem[1 - i])
                    gd[b].wait()
                    sd[b] = pltpu.async_copy(
                        rows[i], acc.at[off_st.at[b]], ssem[i], add=True)
                sd[6].wait()
                sd[7].wait()

            plsc.subcore_barrier()
            pltpu.sync_copy(acc.at[pl.ds(s * stripe, stripe)],
                            s_hbm.at[pl.ds(base + s * stripe, stripe)])

            @pl.when(s == _NSUB - 1)
            def _():
                pltpu.sync_copy(acc.at[pl.ds(_NSUB * stripe, rem)],
                                s_hbm.at[pl.ds(base + _NSUB * stripe, rem)])

            plsc.subcore_barrier()

    return seg(fsrc, foff, kcnt, t0, t1, zz)


@functools.partial(jax.jit, static_argnames=("hs",))
def _counts(dv2d, dr2d, *, hs):
    rows2d = dv2d.shape[0]
    per_w = rows2d // (_NSUB * _NCORE)

    @functools.partial(
        pl.kernel,
        out_type=(jax.ShapeDtypeStruct((_NSUB * _NCORE, 1, hs), F32),
                  jax.ShapeDtypeStruct((_NSUB * _NCORE, 1, hs), F32)),
        mesh=_mesh(),
        compiler_params=_sc_params(),
        scratch_types=[
            pltpu.VMEM((per_w, _EB), I32),
            pltpu.VMEM((hs,), F32),
        ],
    )
    def cnts(dv_hbm, dr_hbm, cv_hbm, cr_hbm, dst_st, hist):
        c = lax.axis_index("c")
        s = lax.axis_index("s")
        wid = s * _NCORE + c
        ones = jnp.ones((16,), F32)
        zeros = jnp.zeros((16,), F32)
        for d_hbm, o_hbm in ((dv_hbm, cv_hbm), (dr_hbm, cr_hbm)):
            @pl.loop(0, hs // 16)
            def _(i):
                hist[pl.ds(i * 16, 16)] = zeros

            pltpu.sync_copy(d_hbm.at[pl.ds(wid * per_w, per_w)], dst_st)

            @pl.loop(0, per_w)
            def _(b):
                for j in range(_EB // 16):
                    d = dst_st[b, pl.ds(j * 16, 16)]
                    plsc.addupdate_scatter(hist, [d], ones)

            pltpu.sync_copy(hist, o_hbm.at[wid, 0])

    return cnts(dv2d, dr2d)


def _combine_body(s0_ref, s1_ref, c_ref, x0_ref, x1_ref, wl_ref, wr_ref,
                  b_ref, *outs, relu, split):
    cnt = jnp.sum(c_ref[...], axis=1)
    cl = jnp.maximum(cnt, 1.0)[:, None]
    wl = wl_ref[...]
    wr = wr_ref[...]
    acc = (jnp.dot(s0_ref[...] / cl, wl[:64], preferred_element_type=F32)
           + jnp.dot(s1_ref[...] / cl, wl[64:], preferred_element_type=F32)
           + jnp.dot(x0_ref[...], wr[:64], preferred_element_type=F32)
           + jnp.dot(x1_ref[...], wr[64:], preferred_element_type=F32)
           + b_ref[...])
    if relu:
        acc = jnp.maximum(acc, 0.0)
    if split:
        outs[0][...] = acc[:, :64]
        outs[1][...] = acc[:, 64:]
    else:
        outs[0][...] = acc


@functools.partial(jax.jit, static_argnames=("relu", "split"))
def _combine(s0, s1, cnt, x0, x1, wl, wr, b, *, relu, split):
    n = s0.shape[0]
    r = 400
    grid = n // r
    nw = cnt.shape[1]
    if split:
        out_shape = (jax.ShapeDtypeStruct((n, 64), F32),
                     jax.ShapeDtypeStruct((n, 64), F32))
        out_specs = (pl.BlockSpec((r, 64), lambda i: (i, 0)),
                     pl.BlockSpec((r, 64), lambda i: (i, 0)))
    else:
        out_shape = jax.ShapeDtypeStruct((n, 128), F32)
        out_specs = pl.BlockSpec((r, 128), lambda i: (i, 0))
    return pl.pallas_call(
        functools.partial(_combine_body, relu=relu, split=split),
        grid=(grid,),
        in_specs=[
            pl.BlockSpec((r, 64), lambda i: (i, 0)),
            pl.BlockSpec((r, 64), lambda i: (i, 0)),
            pl.BlockSpec((r, nw), lambda i: (i, 0)),
            pl.BlockSpec((r, 64), lambda i: (i, 0)),
            pl.BlockSpec((r, 64), lambda i: (i, 0)),
            pl.BlockSpec((128, 128), lambda i: (0, 0)),
            pl.BlockSpec((128, 128), lambda i: (0, 0)),
            pl.BlockSpec((1, 128), lambda i: (0, 0)),
        ],
        out_specs=out_specs,
        out_shape=out_shape,
    )(s0, s1, cnt, x0, x1, wl, wr, b.reshape(1, 128))


def _prep_edges(ei, n_dst):
    src = ei[0].astype(I32)
    dst = ei[1].astype(I32)
    e = src.shape[0]
    ep = -(-e // 32768) * 32768
    fill = jnp.arange(ep - e, dtype=I32)
    psrc = fill % n_dst
    pdst = n_dst + (fill % 64)
    if e % _NSUB == 0 and (ep - e) % _NSUB == 0:
        src = jnp.concatenate(
            [src.reshape(_NSUB, -1), psrc.reshape(_NSUB, -1)], axis=1)
        dst = jnp.concatenate(
            [dst.reshape(_NSUB, -1), pdst.reshape(_NSUB, -1)], axis=1)
    else:
        src = jnp.concatenate([src, psrc])
        dst = jnp.concatenate([dst, pdst])
    return src.reshape(-1, _EB), dst.reshape(-1, _EB)


def kernel(x_person, x_product, edge_index_viewed, edge_index_rev,
           W_l0_v, b0_v, W_r0_v, W_l0_r, b0_r, W_r0_r,
           W_l1_v, b1_v, W_r1_v, W_l1_r, b1_r, W_r1_r):
    np_, nq = x_person.shape[0], x_product.shape[0]
    xp0, xp1 = x_person[:, :64], x_person[:, 64:]
    xq0, xq1 = x_product[:, :64], x_product[:, 64:]
    srcv, dstv = _prep_edges(edge_index_viewed, nq)
    srcr, dstr = _prep_edges(edge_index_rev, np_)
    zz = jnp.zeros((_ACC_ROWS // _NSUB, 64), F32)
    hs = -(-(max(np_, nq) + 64) // 16) * 16

    cv, cr = _counts(dstv, dstr, hs=hs)
    cv = cv[:, 0, :nq].T
    cr = cr[:, 0, :np_].T

    fv, ov, kv, fr, orr, kr = _binedges(srcv, dstv, srcr, dstr, n_dst=nq)

    sv0, sv1, sr0, sr1 = _seglayer(fv, ov, kv, fr, orr, kr,
                                   xp0, xp1, xq0, xq1, zz, n_dst=nq)
    hq0, hq1 = _combine(sv0, sv1, cv, xq0, xq1, W_l0_v, W_r0_v, b0_v,
                        relu=True, split=True)
    hp0, hp1 = _combine(sr0, sr1, cr, xp0, xp1, W_l0_r, W_r0_r, b0_r,
                        relu=True, split=True)

    sv0b, sv1b, sr0b, sr1b = _seglayer(fv, ov, kv, fr, orr, kr,
                                       hp0, hp1, hq0, hq1, zz, n_dst=nq)
    h_prod2 = _combine(sv0b, sv1b, cv, hq0, hq1, W_l1_v, W_r1_v, b1_v,
                       relu=False, split=False)
    h_pers2 = _combine(sr0b, sr1b, cr, hp0, hp1, W_l1_r, W_r1_r, b1_r,
                       relu=False, split=False)
    return (h_pers2, h_prod2)


# speedup vs baseline: 1.2173x; 1.2173x over previous
import dataclasses
import functools

import jax
import jax.numpy as jnp
from jax import lax
from jax.experimental import pallas as pl
from jax.experimental.pallas import tpu as pltpu
from jax.experimental.pallas import tpu_sc as plsc

F32 = jnp.float32
I32 = jnp.int32

_EB = 128
_NSUB = 16
_NCORE = 2
_ACC_ROWS = 25088


def _mesh():
    return plsc.VectorSubcoreMesh(core_axis_name="c", subcore_axis_name="s")


def _sc_params(tc_tiling=True):
    cp = pltpu.CompilerParams()
    fields = pltpu.CompilerParams.__dataclass_fields__
    if "needs_layout_passes" in fields:
        cp = dataclasses.replace(cp, needs_layout_passes=False)
    if not tc_tiling and "use_tc_tiling_on_sc" in fields:
        cp = dataclasses.replace(cp, use_tc_tiling_on_sc=False)
    return cp


@functools.partial(jax.jit, static_argnames=("n_dst",))
def _segsum(src2d, dst2d, t0, t1, zz, *, n_dst):
    rows2d = src2d.shape[0]
    per_tile = rows2d // _NSUB
    half = n_dst // 2
    stripe = (half // _NSUB) // 8 * 8
    rem = half - _NSUB * stripe
    acc_rows = _ACC_ROWS
    assert acc_rows >= half + 1

    chunk = 32
    n_chunks = per_tile // chunk

    @functools.partial(
        pl.kernel,
        out_type=(jax.ShapeDtypeStruct((n_dst, 64), F32),
                  jax.ShapeDtypeStruct((n_dst, 64), F32)),
        mesh=_mesh(),
        compiler_params=_sc_params(tc_tiling=False),
        scratch_types=[
            pltpu.VMEM((chunk, _EB), I32),
            pltpu.VMEM((chunk, _EB), I32),
            pltpu.VMEM((_EB, 64), F32),
            pltpu.VMEM((_EB, 64), F32),
            pltpu.VMEM_SHARED((acc_rows, 64), F32),
            pltpu.SemaphoreType.DMA,
            pltpu.SemaphoreType.DMA,
            pltpu.SemaphoreType.DMA,
            pltpu.SemaphoreType.DMA,
        ],
    )
    def seg(src_hbm, dst_hbm, t0_hbm, t1_hbm, zz_hbm, s0_hbm, s1_hbm,
            src_st, off_st, rows0, rows1, acc, gs0, gs1, ss0, ss1):
        c = lax.axis_index("c")
        s = lax.axis_index("s")
        base = c * half
        zstripe = acc_rows // _NSUB
        rows = (rows0, rows1)
        gsem = (gs0, gs1)
        ssem = (ss0, ss1)

        for t_hbm, s_hbm in ((t0_hbm, s0_hbm), (t1_hbm, s1_hbm)):
            pltpu.sync_copy(zz_hbm, acc.at[pl.ds(s * zstripe, zstripe)])
            plsc.subcore_barrier()

            @pl.loop(0, n_chunks)
            def _(ch):
                row0 = s * per_tile + ch * chunk
                pltpu.sync_copy(src_hbm.at[pl.ds(row0, chunk)], src_st)
                pltpu.sync_copy(dst_hbm.at[pl.ds(row0, chunk)], off_st)

                @pl.loop(0, chunk)
                def _(b):
                    for j in range(_EB // 16):
                        v = off_st[b, pl.ds(j * 16, 16)]
                        o = v - base
                        ok = (o >= 0) & (o < half)
                        off_st[b, pl.ds(j * 16, 16)] = jnp.where(
                            ok, o, half + (v & 63))

                gd = [None] * chunk
                sd = [None] * chunk
                gd[0] = pltpu.async_copy(
                    t_hbm.at[src_st.at[0]], rows[0], gsem[0])
                for b in range(chunk):
                    i = b & 1
                    if b + 1 < chunk:
                        if b >= 1:
                            sd[b - 1].wait()
                        gd[b + 1] = pltpu.async_copy(
                            t_hbm.at[src_st.at[b + 1]], rows[1 - i],
                            gsem[1 - i])
                    gd[b].wait()
                    sd[b] = pltpu.async_copy(
                        rows[i], acc.at[off_st.at[b]], ssem[i], add=True)
                sd[chunk - 2].wait()
                sd[chunk - 1].wait()

            plsc.subcore_barrier()
            pltpu.sync_copy(acc.at[pl.ds(s * stripe, stripe)],
                            s_hbm.at[pl.ds(base + s * stripe, stripe)])

            @pl.when(s == _NSUB - 1)
            def _():
                pltpu.sync_copy(acc.at[pl.ds(_NSUB * stripe, rem)],
                                s_hbm.at[pl.ds(base + _NSUB * stripe, rem)])

            plsc.subcore_barrier()

    return seg(src2d, dst2d, t0, t1, zz)


_CHE = 1024
_CAPC = 35
_BUFC = 2064


@functools.partial(jax.jit, static_argnames=("n_dst",))
def _binedges(sv2d, dv2d, sr2d, dr2d, *, n_dst):
    rows2d = sv2d.shape[0]
    per_tile = rows2d // _NSUB
    half = n_dst // 2

    flat_t = jax.ShapeDtypeStruct((2, _NSUB, 1, _CAPC * _CHE), I32)
    blk_t = jax.ShapeDtypeStruct((2, _NSUB, _CAPC * 8, _EB), I32)
    cnt_t = jax.ShapeDtypeStruct((2, _NSUB, 1, 16), I32)

    @functools.partial(
        pl.kernel,
        out_type=(flat_t, blk_t, cnt_t, flat_t, blk_t, cnt_t),
        mesh=_mesh(),
        compiler_params=_sc_params(tc_tiling=False),
        scratch_types=[
            pltpu.VMEM((32, _EB), I32),
            pltpu.VMEM((32, _EB), I32),
            pltpu.VMEM((_BUFC,), I32),
            pltpu.VMEM((_BUFC,), I32),
            pltpu.VMEM((_BUFC,), I32),
            pltpu.VMEM((_BUFC,), I32),
            pltpu.VMEM((8, _EB), I32),
            pltpu.VMEM((16,), I32),
        ],
    )
    def binker(sv_hbm, dv_hbm, sr_hbm, dr_hbm,
               fv_hbm, ov_hbm, kv_hbm, fr_hbm, orr_hbm, kr_hbm,
               st_s, st_d, bs0, bo0, bs1, bo1, fl2, kst):
        c = lax.axis_index("c")
        s = lax.axis_index("s")
        iota = lax.iota(I32, 16)

        def bin_et(src_hbm, dst_hbm, f_out, o_out, k_out):
            bufs = ((bs0, bo0), (bs1, bo1))

            def flush_chunk(bkt, off_in_buf, k):
                bsrc, boff = bufs[bkt]
                for r in range(8):
                    for j in range(8):
                        fl2[r, pl.ds(j * 16, 16)] = (
                            boff[pl.ds(off_in_buf + r * _EB + j * 16, 16)])
                pltpu.sync_copy(
                    bsrc.at[pl.ds(off_in_buf, _CHE)],
                    f_out.at[bkt, s, 0, pl.ds(k * _CHE, _CHE)])
                pltpu.sync_copy(fl2, o_out.at[bkt, s, pl.ds(k * 8, 8)])

            def maybe_flush(bkt):
                def do(args):
                    f, k = args
                    flush_chunk(bkt, 0, k)
                    bsrc, boff = bufs[bkt]
                    for j in range(9):
                        t = bsrc[pl.ds(_CHE + j * 16, 16)]
                        bsrc[pl.ds(j * 16, 16)] = t
                        t2 = boff[pl.ds(_CHE + j * 16, 16)]
                        boff[pl.ds(j * 16, 16)] = t2
                    return (f - _CHE, k + 1)

                def keep(args):
                    return args

                return lambda f, k: lax.cond(f >= _CHE, do, keep, (f, k))

            def chunk_body(ch, carry):
                row0 = s * per_tile + ch * 32
                pltpu.sync_copy(src_hbm.at[pl.ds(row0, 32)], st_s)
                pltpu.sync_copy(dst_hbm.at[pl.ds(row0, 32)], st_d)

                def row_body(r, carry):
                    f0, k0, f1, k1 = carry
                    for j in range(8):
                        sv = st_s[r, pl.ds(j * 16, 16)]
                        dv = st_d[r, pl.ds(j * 16, 16)]
                        m0 = dv < half
                        n0 = jnp.sum(m0.astype(I32), axis=0)
                        plsc.store_compressed(bs0.at[pl.ds(f0, 16)], sv, mask=m0)
                        plsc.store_compressed(bo0.at[pl.ds(f0, 16)], dv, mask=m0)
                        m1 = jnp.logical_not(m0)
                        plsc.store_compressed(bs1.at[pl.ds(f1, 16)], sv, mask=m1)
                        plsc.store_compressed(
                            bo1.at[pl.ds(f1, 16)], dv - half, mask=m1)
                        f0 = f0 + n0
                        f1 = f1 + (16 - n0)
                    f0, k0 = maybe_flush(0)(f0, k0)
                    f1, k1 = maybe_flush(1)(f1, k1)
                    return (f0, k0, f1, k1)

                return lax.fori_loop(0, 32, row_body, carry)

            f0, k0, f1, k1 = lax.fori_loop(
                0, per_tile // 32, chunk_body, (0, 0, 0, 0))

            def drain(bkt, f, k):
                bsrc, boff = bufs[bkt]
                bsrc[pl.ds(f, 16)] = iota + ((f * 37) & 16383)
                boff[pl.ds(f, 16)] = half + ((iota + f) & 63)
                f = (f & ~15) + 16

                def wcond(st):
                    return (st[0] & (_CHE - 1)) != 0

                def wbody(st):
                    fw = st[0]
                    bsrc[pl.ds(fw, 16)] = iota + ((fw * 37) & 16383)
                    boff[pl.ds(fw, 16)] = half + ((iota + fw) & 63)
                    return (fw + 16,)

                f = lax.while_loop(wcond, wbody, (f,))[0]

                def fl(i, kk):
                    flush_chunk(bkt, i * _CHE, kk)
                    return kk + 1

                return lax.fori_loop(0, f // _CHE, fl, k)

            k0 = drain(0, f0, k0)
            k1 = drain(1, f1, k1)
            kst[...] = jnp.broadcast_to(k0, (16,)).astype(I32)
            pltpu.sync_copy(kst, k_out.at[0, s, 0])
            kst[...] = jnp.broadcast_to(k1, (16,)).astype(I32)
            pltpu.sync_copy(kst, k_out.at[1, s, 0])

        @pl.when(c == 0)
        def _():
            bin_et(sv_hbm, dv_hbm, fv_hbm, ov_hbm, kv_hbm)

        @pl.when(c == 1)
        def _():
            bin_et(sr_hbm, dr_hbm, fr_hbm, orr_hbm, kr_hbm)

    return binker(sv2d, dv2d, sr2d, dr2d)


@functools.partial(jax.jit, static_argnames=("n_dst",))
def _segsum_b(fsrc, foff, kcnt, t0, t1, zz, *, n_dst):
    half = n_dst // 2
    stripe = (half // _NSUB) // 8 * 8
    rem = half - _NSUB * stripe
    acc_rows = _ACC_ROWS
    assert acc_rows >= half + 64

    @functools.partial(
        pl.kernel,
        out_type=(jax.ShapeDtypeStruct((n_dst, 64), F32),
                  jax.ShapeDtypeStruct((n_dst, 64), F32)),
        mesh=_mesh(),
        compiler_params=_sc_params(tc_tiling=False),
        scratch_types=[
            pltpu.VMEM((_CHE,), I32),
            pltpu.VMEM((8, _EB), I32),
            pltpu.VMEM((_EB, 64), F32),
            pltpu.VMEM((_EB, 64), F32),
            pltpu.VMEM((16,), I32),
            pltpu.VMEM_SHARED((acc_rows, 64), F32),
            pltpu.SemaphoreType.DMA,
            pltpu.SemaphoreType.DMA,
            pltpu.SemaphoreType.DMA,
            pltpu.SemaphoreType.DMA,
        ],
    )
    def seg(fsrc_hbm, foff_hbm, kcnt_hbm, t0_hbm, t1_hbm, zz_hbm,
            s0_hbm, s1_hbm,
            src_st, off_st, rows0, rows1, kst, acc, gs0, gs1, ss0, ss1):
        c = lax.axis_index("c")
        s = lax.axis_index("s")
        base = c * half
        zstripe = acc_rows // _NSUB
        rows = (rows0, rows1)
        gsem = (gs0, gs1)
        ssem = (ss0, ss1)

        pltpu.sync_copy(kcnt_hbm.at[c, s, 0], kst)
        nck = jnp.max(kst[...], axis=0)

        for t_hbm, s_hbm in ((t0_hbm, s0_hbm), (t1_hbm, s1_hbm)):
            pltpu.sync_copy(zz_hbm, acc.at[pl.ds(s * zstripe, zstripe)])
            plsc.subcore_barrier()

            @pl.loop(0, nck)
            def _(k):
                pltpu.sync_copy(
                    fsrc_hbm.at[c, s, 0, pl.ds(k * _CHE, _CHE)], src_st)
                pltpu.sync_copy(foff_hbm.at[c, s, pl.ds(k * 8, 8)], off_st)

                gd = [None] * 8
                sd = [None] * 8
                gd[0] = pltpu.async_copy(
                    t_hbm.at[src_st.at[pl.ds(0, _EB)]], rows[0], gsem[0])
                for b in range(8):
                    i = b & 1
                    if b + 1 < 8:
                        if b >= 1:
                            sd[b - 1].wait()
                        gd[b + 1] = pltpu.async_copy(
                            t_hbm.at[src_st.at[pl.ds((b + 1) * _EB, _EB)]],
                            rows[1 - i], gsem[1 - i])
                    gd[b].wait()
                    sd[b] = pltpu.async_copy(
                        rows[i], acc.at[off_st.at[b]], ssem[i], add=True)
                sd[6].wait()
                sd[7].wait()

            plsc.subcore_barrier()
            pltpu.sync_copy(acc.at[pl.ds(s * stripe, stripe)],
                            s_hbm.at[pl.ds(base + s * stripe, stripe)])

            @pl.when(s == _NSUB - 1)
            def _():
                pltpu.sync_copy(acc.at[pl.ds(_NSUB * stripe, rem)],
                                s_hbm.at[pl.ds(base + _NSUB * stripe, rem)])

            plsc.subcore_barrier()

    return seg(fsrc, foff, kcnt, t0, t1, zz)


@functools.partial(jax.jit, static_argnames=("hs",))
def _counts(dv2d, dr2d, *, hs):
    rows2d = dv2d.shape[0]
    per_w = rows2d // (_NSUB * _NCORE)

    @functools.partial(
        pl.kernel,
        out_type=(jax.ShapeDtypeStruct((_NSUB * _NCORE, 1, hs), F32),
                  jax.ShapeDtypeStruct((_NSUB * _NCORE, 1, hs), F32)),
        mesh=_mesh(),
        compiler_params=_sc_params(),
        scratch_types=[
            pltpu.VMEM((per_w, _EB), I32),
            pltpu.VMEM((hs,), F32),
        ],
    )
    def cnts(dv_hbm, dr_hbm, cv_hbm, cr_hbm, dst_st, hist):
        c = lax.axis_index("c")
        s = lax.axis_index("s")
        wid = s * _NCORE + c
        ones = jnp.ones((16,), F32)
        zeros = jnp.zeros((16,), F32)
        for d_hbm, o_hbm in ((dv_hbm, cv_hbm), (dr_hbm, cr_hbm)):
            @pl.loop(0, hs // 16)
            def _(i):
                hist[pl.ds(i * 16, 16)] = zeros

            pltpu.sync_copy(d_hbm.at[pl.ds(wid * per_w, per_w)], dst_st)

            @pl.loop(0, per_w)
            def _(b):
                for j in range(_EB // 16):
                    d = dst_st[b, pl.ds(j * 16, 16)]
                    plsc.addupdate_scatter(hist, [d], ones)

            pltpu.sync_copy(hist, o_hbm.at[wid, 0])

    return cnts(dv2d, dr2d)


def _combine_body(s0_ref, s1_ref, c_ref, x0_ref, x1_ref, wl_ref, wr_ref,
                  b_ref, *outs, relu, split):
    cnt = jnp.sum(c_ref[...], axis=1)
    cl = jnp.maximum(cnt, 1.0)[:, None]
    wl = wl_ref[...]
    wr = wr_ref[...]
    acc = (jnp.dot(s0_ref[...] / cl, wl[:64], preferred_element_type=F32)
           + jnp.dot(s1_ref[...] / cl, wl[64:], preferred_element_type=F32)
           + jnp.dot(x0_ref[...], wr[:64], preferred_element_type=F32)
           + jnp.dot(x1_ref[...], wr[64:], preferred_element_type=F32)
           + b_ref[...])
    if relu:
        acc = jnp.maximum(acc, 0.0)
    if split:
        outs[0][...] = acc[:, :64]
        outs[1][...] = acc[:, 64:]
    else:
        outs[0][...] = acc


@functools.partial(jax.jit, static_argnames=("relu", "split"))
def _combine(s0, s1, cnt, x0, x1, wl, wr, b, *, relu, split):
    n = s0.shape[0]
    r = 400
    grid = n // r
    nw = cnt.shape[1]
    if split:
        out_shape = (jax.ShapeDtypeStruct((n, 64), F32),
                     jax.ShapeDtypeStruct((n, 64), F32))
        out_specs = (pl.BlockSpec((r, 64), lambda i: (i, 0)),
                     pl.BlockSpec((r, 64), lambda i: (i, 0)))
    else:
        out_shape = jax.ShapeDtypeStruct((n, 128), F32)
        out_specs = pl.BlockSpec((r, 128), lambda i: (i, 0))
    return pl.pallas_call(
        functools.partial(_combine_body, relu=relu, split=split),
        grid=(grid,),
        in_specs=[
            pl.BlockSpec((r, 64), lambda i: (i, 0)),
            pl.BlockSpec((r, 64), lambda i: (i, 0)),
            pl.BlockSpec((r, nw), lambda i: (i, 0)),
            pl.BlockSpec((r, 64), lambda i: (i, 0)),
            pl.BlockSpec((r, 64), lambda i: (i, 0)),
            pl.BlockSpec((128, 128), lambda i: (0, 0)),
            pl.BlockSpec((128, 128), lambda i: (0, 0)),
            pl.BlockSpec((1, 128), lambda i: (0, 0)),
        ],
        out_specs=out_specs,
        out_shape=out_shape,
    )(s0, s1, cnt, x0, x1, wl, wr, b.reshape(1, 128))


def _prep_edges(ei, n_dst):
    src = ei[0].astype(I32)
    dst = ei[1].astype(I32)
    e = src.shape[0]
    ep = -(-e // 32768) * 32768
    fill = jnp.arange(ep - e, dtype=I32)
    psrc = fill % n_dst
    pdst = n_dst + (fill % 64)
    if e % _NSUB == 0 and (ep - e) % _NSUB == 0:
        src = jnp.concatenate(
            [src.reshape(_NSUB, -1), psrc.reshape(_NSUB, -1)], axis=1)
        dst = jnp.concatenate(
            [dst.reshape(_NSUB, -1), pdst.reshape(_NSUB, -1)], axis=1)
    else:
        src = jnp.concatenate([src, psrc])
        dst = jnp.concatenate([dst, pdst])
    return src.reshape(-1, _EB), dst.reshape(-1, _EB)


def kernel(x_person, x_product, edge_index_viewed, edge_index_rev,
           W_l0_v, b0_v, W_r0_v, W_l0_r, b0_r, W_r0_r,
           W_l1_v, b1_v, W_r1_v, W_l1_r, b1_r, W_r1_r):
    np_, nq = x_person.shape[0], x_product.shape[0]
    xp0, xp1 = x_person[:, :64], x_person[:, 64:]
    xq0, xq1 = x_product[:, :64], x_product[:, 64:]
    srcv, dstv = _prep_edges(edge_index_viewed, nq)
    srcr, dstr = _prep_edges(edge_index_rev, np_)
    zz = jnp.zeros((_ACC_ROWS // _NSUB, 64), F32)
    hs = -(-(max(np_, nq) + 64) // 16) * 16

    cv, cr = _counts(dstv, dstr, hs=hs)
    cv = cv[:, 0, :nq].T
    cr = cr[:, 0, :np_].T

    fv, ov, kv, fr, orr, kr = _binedges(srcv, dstv, srcr, dstr, n_dst=nq)

    sv0, sv1 = _segsum_b(fv, ov, kv, xp0, xp1, zz, n_dst=nq)
    sr0, sr1 = _segsum_b(fr, orr, kr, xq0, xq1, zz, n_dst=np_)
    hq0, hq1 = _combine(sv0, sv1, cv, xq0, xq1, W_l0_v, W_r0_v, b0_v,
                        relu=True, split=True)
    hp0, hp1 = _combine(sr0, sr1, cr, xp0, xp1, W_l0_r, W_r0_r, b0_r,
                        relu=True, split=True)

    sv0b, sv1b = _segsum_b(fv, ov, kv, hp0, hp1, zz, n_dst=nq)
    sr0b, sr1b = _segsum_b(fr, orr, kr, hq0, hq1, zz, n_dst=np_)
    h_prod2 = _combine(sv0b, sv1b, cv, hq0, hq1, W_l1_v, W_r1_v, b1_v,
                       relu=False, split=False)
    h_pers2 = _combine(sr0b, sr1b, cr, hp0, hp1, W_l1_r, W_r1_r, b1_r,
                       relu=False, split=False)
    return (h_pers2, h_prod2)


# speedup vs baseline: 1.2865x; 1.0569x over previous
import dataclasses
import functools

import jax
import jax.numpy as jnp
from jax import lax
from jax.experimental import pallas as pl
from jax.experimental.pallas import tpu as pltpu
from jax.experimental.pallas import tpu_sc as plsc

F32 = jnp.float32
I32 = jnp.int32

_EB = 128
_NSUB = 16
_NCORE = 2
_ACC_ROWS = 25088


def _mesh():
    return plsc.VectorSubcoreMesh(core_axis_name="c", subcore_axis_name="s")


def _sc_params(tc_tiling=True):
    cp = pltpu.CompilerParams()
    fields = pltpu.CompilerParams.__dataclass_fields__
    if "needs_layout_passes" in fields:
        cp = dataclasses.replace(cp, needs_layout_passes=False)
    if not tc_tiling and "use_tc_tiling_on_sc" in fields:
        cp = dataclasses.replace(cp, use_tc_tiling_on_sc=False)
    return cp


@functools.partial(jax.jit, static_argnames=("n_dst",))
def _segsum(src2d, dst2d, t0, t1, zz, *, n_dst):
    rows2d = src2d.shape[0]
    per_tile = rows2d // _NSUB
    half = n_dst // 2
    stripe = (half // _NSUB) // 8 * 8
    rem = half - _NSUB * stripe
    acc_rows = _ACC_ROWS
    assert acc_rows >= half + 1

    chunk = 32
    n_chunks = per_tile // chunk

    @functools.partial(
        pl.kernel,
        out_type=(jax.ShapeDtypeStruct((n_dst, 64), F32),
                  jax.ShapeDtypeStruct((n_dst, 64), F32)),
        mesh=_mesh(),
        compiler_params=_sc_params(tc_tiling=False),
        scratch_types=[
            pltpu.VMEM((chunk, _EB), I32),
            pltpu.VMEM((chunk, _EB), I32),
            pltpu.VMEM((_EB, 64), F32),
            pltpu.VMEM((_EB, 64), F32),
            pltpu.VMEM_SHARED((acc_rows, 64), F32),
            pltpu.SemaphoreType.DMA,
            pltpu.SemaphoreType.DMA,
            pltpu.SemaphoreType.DMA,
            pltpu.SemaphoreType.DMA,
        ],
    )
    def seg(src_hbm, dst_hbm, t0_hbm, t1_hbm, zz_hbm, s0_hbm, s1_hbm,
            src_st, off_st, rows0, rows1, acc, gs0, gs1, ss0, ss1):
        c = lax.axis_index("c")
        s = lax.axis_index("s")
        base = c * half
        zstripe = acc_rows // _NSUB
        rows = (rows0, rows1)
        gsem = (gs0, gs1)
        ssem = (ss0, ss1)

        for t_hbm, s_hbm in ((t0_hbm, s0_hbm), (t1_hbm, s1_hbm)):
            pltpu.sync_copy(zz_hbm, acc.at[pl.ds(s * zstripe, zstripe)])
            plsc.subcore_barrier()

            @pl.loop(0, n_chunks)
            def _(ch):
                row0 = s * per_tile + ch * chunk
                pltpu.sync_copy(src_hbm.at[pl.ds(row0, chunk)], src_st)
                pltpu.sync_copy(dst_hbm.at[pl.ds(row0, chunk)], off_st)

                @pl.loop(0, chunk)
                def _(b):
                    for j in range(_EB // 16):
                        v = off_st[b, pl.ds(j * 16, 16)]
                        o = v - base
                        ok = (o >= 0) & (o < half)
                        off_st[b, pl.ds(j * 16, 16)] = jnp.where(
                            ok, o, half + (v & 63))

                gd = [None] * chunk
                sd = [None] * chunk
                gd[0] = pltpu.async_copy(
                    t_hbm.at[src_st.at[0]], rows[0], gsem[0])
                for b in range(chunk):
                    i = b & 1
                    if b + 1 < chunk:
                        if b >= 1:
                            sd[b - 1].wait()
                        gd[b + 1] = pltpu.async_copy(
                            t_hbm.at[src_st.at[b + 1]], rows[1 - i],
                            gsem[1 - i])
                    gd[b].wait()
                    sd[b] = pltpu.async_copy(
                        rows[i], acc.at[off_st.at[b]], ssem[i], add=True)
                sd[chunk - 2].wait()
                sd[chunk - 1].wait()

            plsc.subcore_barrier()
            pltpu.sync_copy(acc.at[pl.ds(s * stripe, stripe)],
                            s_hbm.at[pl.ds(base + s * stripe, stripe)])

            @pl.when(s == _NSUB - 1)
            def _():
                pltpu.sync_copy(acc.at[pl.ds(_NSUB * stripe, rem)],
                                s_hbm.at[pl.ds(base + _NSUB * stripe, rem)])

            plsc.subcore_barrier()

    return seg(src2d, dst2d, t0, t1, zz)


_CHE = 1024
_CAPC = 35
_BUFC = 2064


@functools.partial(jax.jit, static_argnames=("n_dst",))
def _binedges(sv2d, dv2d, sr2d, dr2d, *, n_dst):
    rows2d = sv2d.shape[0]
    per_tile = rows2d // _NSUB
    half = n_dst // 2

    flat_t = jax.ShapeDtypeStruct((2, _NSUB, 1, _CAPC * _CHE), I32)
    blk_t = jax.ShapeDtypeStruct((2, _NSUB, _CAPC * 8, _EB), I32)
    cnt_t = jax.ShapeDtypeStruct((2, _NSUB, 1, 16), I32)

    @functools.partial(
        pl.kernel,
        out_type=(flat_t, blk_t, cnt_t, flat_t, blk_t, cnt_t),
        mesh=_mesh(),
        compiler_params=_sc_params(tc_tiling=False),
        scratch_types=[
            pltpu.VMEM((32, _EB), I32),
            pltpu.VMEM((32, _EB), I32),
            pltpu.VMEM((_BUFC,), I32),
            pltpu.VMEM((_BUFC,), I32),
            pltpu.VMEM((_BUFC,), I32),
            pltpu.VMEM((_BUFC,), I32),
            pltpu.VMEM((8, _EB), I32),
            pltpu.VMEM((16,), I32),
        ],
    )
    def binker(sv_hbm, dv_hbm, sr_hbm, dr_hbm,
               fv_hbm, ov_hbm, kv_hbm, fr_hbm, orr_hbm, kr_hbm,
               st_s, st_d, bs0, bo0, bs1, bo1, fl2, kst):
        c = lax.axis_index("c")
        s = lax.axis_index("s")
        iota = lax.iota(I32, 16)

        def bin_et(src_hbm, dst_hbm, f_out, o_out, k_out):
            bufs = ((bs0, bo0), (bs1, bo1))

            def flush_chunk(bkt, off_in_buf, k):
                bsrc, boff = bufs[bkt]
                for r in range(8):
                    for j in range(8):
                        fl2[r, pl.ds(j * 16, 16)] = (
                            boff[pl.ds(off_in_buf + r * _EB + j * 16, 16)])
                pltpu.sync_copy(
                    bsrc.at[pl.ds(off_in_buf, _CHE)],
                    f_out.at[bkt, s, 0, pl.ds(k * _CHE, _CHE)])
                pltpu.sync_copy(fl2, o_out.at[bkt, s, pl.ds(k * 8, 8)])

            def maybe_flush(bkt):
                def do(args):
                    f, k = args
                    flush_chunk(bkt, 0, k)
                    bsrc, boff = bufs[bkt]
                    for j in range(9):
                        t = bsrc[pl.ds(_CHE + j * 16, 16)]
                        bsrc[pl.ds(j * 16, 16)] = t
                        t2 = boff[pl.ds(_CHE + j * 16, 16)]
                        boff[pl.ds(j * 16, 16)] = t2
                    return (f - _CHE, k + 1)

                def keep(args):
                    return args

                return lambda f, k: lax.cond(f >= _CHE, do, keep, (f, k))

            def chunk_body(ch, carry):
                row0 = s * per_tile + ch * 32
                pltpu.sync_copy(src_hbm.at[pl.ds(row0, 32)], st_s)
                pltpu.sync_copy(dst_hbm.at[pl.ds(row0, 32)], st_d)

                def row_body(r, carry):
                    f0, k0, f1, k1 = carry
                    for j in range(8):
                        sv = st_s[r, pl.ds(j * 16, 16)]
                        dv = st_d[r, pl.ds(j * 16, 16)]
                        m0 = dv < half
                        n0 = jnp.sum(m0.astype(I32), axis=0)
                        plsc.store_compressed(bs0.at[pl.ds(f0, 16)], sv, mask=m0)
                        plsc.store_compressed(bo0.at[pl.ds(f0, 16)], dv, mask=m0)
                        m1 = jnp.logical_not(m0)
                        plsc.store_compressed(bs1.at[pl.ds(f1, 16)], sv, mask=m1)
                        plsc.store_compressed(
                            bo1.at[pl.ds(f1, 16)], dv - half, mask=m1)
                        f0 = f0 + n0
                        f1 = f1 + (16 - n0)
                    f0, k0 = maybe_flush(0)(f0, k0)
                    f1, k1 = maybe_flush(1)(f1, k1)
                    return (f0, k0, f1, k1)

                return lax.fori_loop(0, 32, row_body, carry)

            f0, k0, f1, k1 = lax.fori_loop(
                0, per_tile // 32, chunk_body, (0, 0, 0, 0))

            def drain(bkt, f, k):
                bsrc, boff = bufs[bkt]
                bsrc[pl.ds(f, 16)] = iota + ((f * 37) & 16383)
                boff[pl.ds(f, 16)] = half + ((iota + f) & 63)
                f = (f & ~15) + 16

                def wcond(st):
                    return (st[0] & (_CHE - 1)) != 0

                def wbody(st):
                    fw = st[0]
                    bsrc[pl.ds(fw, 16)] = iota + ((fw * 37) & 16383)
                    boff[pl.ds(fw, 16)] = half + ((iota + fw) & 63)
                    return (fw + 16,)

                f = lax.while_loop(wcond, wbody, (f,))[0]

                def fl(i, kk):
                    flush_chunk(bkt, i * _CHE, kk)
                    return kk + 1

                return lax.fori_loop(0, f // _CHE, fl, k)

            k0 = drain(0, f0, k0)
            k1 = drain(1, f1, k1)
            kst[...] = jnp.broadcast_to(k0, (16,)).astype(I32)
            pltpu.sync_copy(kst, k_out.at[0, s, 0])
            kst[...] = jnp.broadcast_to(k1, (16,)).astype(I32)
            pltpu.sync_copy(kst, k_out.at[1, s, 0])

        @pl.when(c == 0)
        def _():
            bin_et(sv_hbm, dv_hbm, fv_hbm, ov_hbm, kv_hbm)

        @pl.when(c == 1)
        def _():
            bin_et(sr_hbm, dr_hbm, fr_hbm, orr_hbm, kr_hbm)

    return binker(sv2d, dv2d, sr2d, dr2d)


@functools.partial(jax.jit, static_argnames=("n_dst",))
def _segsum_b(fsrc, foff, kcnt, t0, t1, zz, *, n_dst):
    half = n_dst // 2
    stripe = (half // _NSUB) // 8 * 8
    rem = half - _NSUB * stripe
    acc_rows = _ACC_ROWS
    assert acc_rows >= half + 64

    @functools.partial(
        pl.kernel,
        out_type=(jax.ShapeDtypeStruct((n_dst, 64), F32),
                  jax.ShapeDtypeStruct((n_dst, 64), F32)),
        mesh=_mesh(),
        compiler_params=_sc_params(tc_tiling=False),
        scratch_types=[
            pltpu.VMEM((_CHE,), I32),
            pltpu.VMEM((8, _EB), I32),
            pltpu.VMEM((_EB, 64), F32),
            pltpu.VMEM((_EB, 64), F32),
            pltpu.VMEM((16,), I32),
            pltpu.VMEM_SHARED((acc_rows, 64), F32),
            pltpu.SemaphoreType.DMA,
            pltpu.SemaphoreType.DMA,
            pltpu.SemaphoreType.DMA,
            pltpu.SemaphoreType.DMA,
        ],
    )
    def seg(fsrc_hbm, foff_hbm, kcnt_hbm, t0_hbm, t1_hbm, zz_hbm,
            s0_hbm, s1_hbm,
            src_st, off_st, rows0, rows1, kst, acc, gs0, gs1, ss0, ss1):
        c = lax.axis_index("c")
        s = lax.axis_index("s")
        base = c * half
        zstripe = acc_rows // _NSUB
        rows = (rows0, rows1)
        gsem = (gs0, gs1)
        ssem = (ss0, ss1)

        pltpu.sync_copy(kcnt_hbm.at[c, s, 0], kst)
        nck = jnp.max(kst[...], axis=0)

        for t_hbm, s_hbm in ((t0_hbm, s0_hbm), (t1_hbm, s1_hbm)):
            pltpu.sync_copy(zz_hbm, acc.at[pl.ds(s * zstripe, zstripe)])
            plsc.subcore_barrier()

            @pl.loop(0, nck)
            def _(k):
                pltpu.sync_copy(
                    fsrc_hbm.at[c, s, 0, pl.ds(k * _CHE, _CHE)], src_st)
                pltpu.sync_copy(foff_hbm.at[c, s, pl.ds(k * 8, 8)], off_st)

                gd = [None] * 8
                sd = [None] * 8
                gd[0] = pltpu.async_copy(
                    t_hbm.at[src_st.at[pl.ds(0, _EB)]], rows[0], gsem[0])
                for b in range(8):
                    i = b & 1
                    if b + 1 < 8:
                        if b >= 1:
                            sd[b - 1].wait()
                        gd[b + 1] = pltpu.async_copy(
                            t_hbm.at[src_st.at[pl.ds((b + 1) * _EB, _EB)]],
                            rows[1 - i], gsem[1 - i])
                    gd[b].wait()
                    sd[b] = pltpu.async_copy(
                        rows[i], acc.at[off_st.at[b]], ssem[i], add=True)
                sd[6].wait()
                sd[7].wait()

            plsc.subcore_barrier()
            pltpu.sync_copy(acc.at[pl.ds(s * stripe, stripe)],
                            s_hbm.at[pl.ds(base + s * stripe, stripe)])

            @pl.when(s == _NSUB - 1)
            def _():
                pltpu.sync_copy(acc.at[pl.ds(_NSUB * stripe, rem)],
                                s_hbm.at[pl.ds(base + _NSUB * stripe, rem)])

            plsc.subcore_barrier()

    return seg(fsrc, foff, kcnt, t0, t1, zz)


@functools.partial(jax.jit, static_argnames=("hs",))
def _counts(dv2d, dr2d, *, hs):
    rows2d = dv2d.shape[0]
    per_w = rows2d // (_NSUB * _NCORE)

    @functools.partial(
        pl.kernel,
        out_type=(jax.ShapeDtypeStruct((_NSUB * _NCORE, 1, hs), F32),
                  jax.ShapeDtypeStruct((_NSUB * _NCORE, 1, hs), F32)),
        mesh=_mesh(),
        compiler_params=_sc_params(),
        scratch_types=[
            pltpu.VMEM((per_w, _EB), I32),
            pltpu.VMEM((hs,), F32),
        ],
    )
    def cnts(dv_hbm, dr_hbm, cv_hbm, cr_hbm, dst_st, hist):
        c = lax.axis_index("c")
        s = lax.axis_index("s")
        wid = s * _NCORE + c
        ones = jnp.ones((16,), F32)
        zeros = jnp.zeros((16,), F32)
        for d_hbm, o_hbm in ((dv_hbm, cv_hbm), (dr_hbm, cr_hbm)):
            @pl.loop(0, hs // 16)
            def _(i):
                hist[pl.ds(i * 16, 16)] = zeros

            pltpu.sync_copy(d_hbm.at[pl.ds(wid * per_w, per_w)], dst_st)

            @pl.loop(0, per_w)
            def _(b):
                for j in range(_EB // 16):
                    d = dst_st[b, pl.ds(j * 16, 16)]
                    plsc.addupdate_scatter(hist, [d], ones)

            pltpu.sync_copy(hist, o_hbm.at[wid, 0])

    return cnts(dv2d, dr2d)


def _combine_body(s0_ref, s1_ref, c_ref, x0_ref, x1_ref, wl_ref, wr_ref,
                  b_ref, *outs, relu, split):
    cnt = jnp.sum(c_ref[...], axis=1)
    cl = jnp.maximum(cnt, 1.0)[:, None]
    wl = wl_ref[...]
    wr = wr_ref[...]
    acc = (jnp.dot(s0_ref[...] / cl, wl[:64], preferred_element_type=F32)
           + jnp.dot(s1_ref[...] / cl, wl[64:], preferred_element_type=F32)
           + jnp.dot(x0_ref[...], wr[:64], preferred_element_type=F32)
           + jnp.dot(x1_ref[...], wr[64:], preferred_element_type=F32)
           + b_ref[...])
    if relu:
        acc = jnp.maximum(acc, 0.0)
    if split:
        outs[0][...] = acc[:, :64]
        outs[1][...] = acc[:, 64:]
    else:
        outs[0][...] = acc


@functools.partial(jax.jit, static_argnames=("relu", "split"))
def _combine(s0, s1, cnt, x0, x1, wl, wr, b, *, relu, split):
    n = s0.shape[0]
    r = 2000
    grid = n // r
    nw = cnt.shape[1]
    if split:
        out_shape = (jax.ShapeDtypeStruct((n, 64), F32),
                     jax.ShapeDtypeStruct((n, 64), F32))
        out_specs = (pl.BlockSpec((r, 64), lambda i: (i, 0)),
                     pl.BlockSpec((r, 64), lambda i: (i, 0)))
    else:
        out_shape = jax.ShapeDtypeStruct((n, 128), F32)
        out_specs = pl.BlockSpec((r, 128), lambda i: (i, 0))
    return pl.pallas_call(
        functools.partial(_combine_body, relu=relu, split=split),
        grid=(grid,),
        in_specs=[
            pl.BlockSpec((r, 64), lambda i: (i, 0)),
            pl.BlockSpec((r, 64), lambda i: (i, 0)),
            pl.BlockSpec((r, nw), lambda i: (i, 0)),
            pl.BlockSpec((r, 64), lambda i: (i, 0)),
            pl.BlockSpec((r, 64), lambda i: (i, 0)),
            pl.BlockSpec((128, 128), lambda i: (0, 0)),
            pl.BlockSpec((128, 128), lambda i: (0, 0)),
            pl.BlockSpec((1, 128), lambda i: (0, 0)),
        ],
        out_specs=out_specs,
        out_shape=out_shape,
    )(s0, s1, cnt, x0, x1, wl, wr, b.reshape(1, 128))


def _prep_edges(ei, n_dst):
    src = ei[0].astype(I32)
    dst = ei[1].astype(I32)
    e = src.shape[0]
    ep = -(-e // 32768) * 32768
    fill = jnp.arange(ep - e, dtype=I32)
    psrc = fill % n_dst
    pdst = n_dst + (fill % 64)
    if e % _NSUB == 0 and (ep - e) % _NSUB == 0:
        src = jnp.concatenate(
            [src.reshape(_NSUB, -1), psrc.reshape(_NSUB, -1)], axis=1)
        dst = jnp.concatenate(
            [dst.reshape(_NSUB, -1), pdst.reshape(_NSUB, -1)], axis=1)
    else:
        src = jnp.concatenate([src, psrc])
        dst = jnp.concatenate([dst, pdst])
    return src.reshape(-1, _EB), dst.reshape(-1, _EB)


def kernel(x_person, x_product, edge_index_viewed, edge_index_rev,
           W_l0_v, b0_v, W_r0_v, W_l0_r, b0_r, W_r0_r,
           W_l1_v, b1_v, W_r1_v, W_l1_r, b1_r, W_r1_r):
    np_, nq = x_person.shape[0], x_product.shape[0]
    xp0, xp1 = x_person[:, :64], x_person[:, 64:]
    xq0, xq1 = x_product[:, :64], x_product[:, 64:]
    srcv, dstv = _prep_edges(edge_index_viewed, nq)
    srcr, dstr = _prep_edges(edge_index_rev, np_)
    zz = jnp.zeros((_ACC_ROWS // _NSUB, 64), F32)
    hs = -(-(max(np_, nq) + 64) // 16) * 16

    cv, cr = _counts(dstv, dstr, hs=hs)
    cv = cv[:, 0, :nq].T
    cr = cr[:, 0, :np_].T

    fv, ov, kv, fr, orr, kr = _binedges(srcv, dstv, srcr, dstr, n_dst=nq)

    sv0, sv1 = _segsum_b(fv, ov, kv, xp0, xp1, zz, n_dst=nq)
    sr0, sr1 = _segsum_b(fr, orr, kr, xq0, xq1, zz, n_dst=np_)
    hp0, hp1 = _combine(sr0, sr1, cr, xp0, xp1, W_l0_r, W_r0_r, b0_r,
                        relu=True, split=True)
    hq0, hq1 = _combine(sv0, sv1, cv, xq0, xq1, W_l0_v, W_r0_v, b0_v,
                        relu=True, split=True)

    sv0b, sv1b = _segsum_b(fv, ov, kv, hp0, hp1, zz, n_dst=nq)
    sr0b, sr1b = _segsum_b(fr, orr, kr, hq0, hq1, zz, n_dst=np_)
    h_prod2 = _combine(sv0b, sv1b, cv, hq0, hq1, W_l1_v, W_r1_v, b1_v,
                       relu=False, split=False)
    h_pers2 = _combine(sr0b, sr1b, cr, hp0, hp1, W_l1_r, W_r1_r, b1_r,
                       relu=False, split=False)
    return (h_pers2, h_prod2)


# speedup vs baseline: 1.3192x; 1.0254x over previous
import dataclasses
import functools

import jax
import jax.numpy as jnp
from jax import lax
from jax.experimental import pallas as pl
from jax.experimental.pallas import tpu as pltpu
from jax.experimental.pallas import tpu_sc as plsc

F32 = jnp.float32
I32 = jnp.int32

_EB = 128
_NSUB = 16
_NCORE = 2
_ACC_ROWS = 25088


def _mesh():
    return plsc.VectorSubcoreMesh(core_axis_name="c", subcore_axis_name="s")


def _sc_params(tc_tiling=True):
    cp = pltpu.CompilerParams()
    fields = pltpu.CompilerParams.__dataclass_fields__
    if "needs_layout_passes" in fields:
        cp = dataclasses.replace(cp, needs_layout_passes=False)
    if not tc_tiling and "use_tc_tiling_on_sc" in fields:
        cp = dataclasses.replace(cp, use_tc_tiling_on_sc=False)
    return cp


@functools.partial(jax.jit, static_argnames=("n_dst",))
def _segsum(src2d, dst2d, t0, t1, zz, *, n_dst):
    rows2d = src2d.shape[0]
    per_tile = rows2d // _NSUB
    half = n_dst // 2
    stripe = (half // _NSUB) // 8 * 8
    rem = half - _NSUB * stripe
    acc_rows = _ACC_ROWS
    assert acc_rows >= half + 1

    chunk = 32
    n_chunks = per_tile // chunk

    @functools.partial(
        pl.kernel,
        out_type=(jax.ShapeDtypeStruct((n_dst, 64), F32),
                  jax.ShapeDtypeStruct((n_dst, 64), F32)),
        mesh=_mesh(),
        compiler_params=_sc_params(tc_tiling=False),
        scratch_types=[
            pltpu.VMEM((chunk, _EB), I32),
            pltpu.VMEM((chunk, _EB), I32),
            pltpu.VMEM((_EB, 64), F32),
            pltpu.VMEM((_EB, 64), F32),
            pltpu.VMEM_SHARED((acc_rows, 64), F32),
            pltpu.SemaphoreType.DMA,
            pltpu.SemaphoreType.DMA,
            pltpu.SemaphoreType.DMA,
            pltpu.SemaphoreType.DMA,
        ],
    )
    def seg(src_hbm, dst_hbm, t0_hbm, t1_hbm, zz_hbm, s0_hbm, s1_hbm,
            src_st, off_st, rows0, rows1, acc, gs0, gs1, ss0, ss1):
        c = lax.axis_index("c")
        s = lax.axis_index("s")
        base = c * half
        zstripe = acc_rows // _NSUB
        rows = (rows0, rows1)
        gsem = (gs0, gs1)
        ssem = (ss0, ss1)

        for t_hbm, s_hbm in ((t0_hbm, s0_hbm), (t1_hbm, s1_hbm)):
            pltpu.sync_copy(zz_hbm, acc.at[pl.ds(s * zstripe, zstripe)])
            plsc.subcore_barrier()

            @pl.loop(0, n_chunks)
            def _(ch):
                row0 = s * per_tile + ch * chunk
                pltpu.sync_copy(src_hbm.at[pl.ds(row0, chunk)], src_st)
                pltpu.sync_copy(dst_hbm.at[pl.ds(row0, chunk)], off_st)

                @pl.loop(0, chunk)
                def _(b):
                    for j in range(_EB // 16):
                        v = off_st[b, pl.ds(j * 16, 16)]
                        o = v - base
                        ok = (o >= 0) & (o < half)
                        off_st[b, pl.ds(j * 16, 16)] = jnp.where(
                            ok, o, half + (v & 63))

                gd = [None] * chunk
                sd = [None] * chunk
                gd[0] = pltpu.async_copy(
                    t_hbm.at[src_st.at[0]], rows[0], gsem[0])
                for b in range(chunk):
                    i = b & 1
                    if b + 1 < chunk:
                        if b >= 1:
                            sd[b - 1].wait()
                        gd[b + 1] = pltpu.async_copy(
                            t_hbm.at[src_st.at[b + 1]], rows[1 - i],
                            gsem[1 - i])
                    gd[b].wait()
                    sd[b] = pltpu.async_copy(
                        rows[i], acc.at[off_st.at[b]], ssem[i], add=True)
                sd[chunk - 2].wait()
                sd[chunk - 1].wait()

            plsc.subcore_barrier()
            pltpu.sync_copy(acc.at[pl.ds(s * stripe, stripe)],
                            s_hbm.at[pl.ds(base + s * stripe, stripe)])

            @pl.when(s == _NSUB - 1)
            def _():
                pltpu.sync_copy(acc.at[pl.ds(_NSUB * stripe, rem)],
                                s_hbm.at[pl.ds(base + _NSUB * stripe, rem)])

            plsc.subcore_barrier()

    return seg(src2d, dst2d, t0, t1, zz)


_CHE = 1024
_CAPC = 35
_BUFC = 2064


@functools.partial(jax.jit, static_argnames=("n_dst", "hs"))
def _binedges(sv2d, dv2d, sr2d, dr2d, *, n_dst, hs):
    rows2d = sv2d.shape[0]
    per_tile = rows2d // _NSUB
    half = n_dst // 2

    flat_t = jax.ShapeDtypeStruct((2, _NSUB, 1, _CAPC * _CHE), I32)
    blk_t = jax.ShapeDtypeStruct((2, _NSUB, _CAPC * 8, _EB), I32)
    cnt_t = jax.ShapeDtypeStruct((2, _NSUB, 1, 16), I32)
    hist_t = jax.ShapeDtypeStruct((_NSUB, 1, hs), F32)

    @functools.partial(
        pl.kernel,
        out_type=(flat_t, blk_t, cnt_t, hist_t, flat_t, blk_t, cnt_t, hist_t),
        mesh=_mesh(),
        compiler_params=_sc_params(tc_tiling=False),
        scratch_types=[
            pltpu.VMEM((32, _EB), I32),
            pltpu.VMEM((32, _EB), I32),
            pltpu.VMEM((_BUFC,), I32),
            pltpu.VMEM((_BUFC,), I32),
            pltpu.VMEM((_BUFC,), I32),
            pltpu.VMEM((_BUFC,), I32),
            pltpu.VMEM((8, _EB), I32),
            pltpu.VMEM((16,), I32),
            pltpu.VMEM((hs,), F32),
        ],
    )
    def binker(sv_hbm, dv_hbm, sr_hbm, dr_hbm,
               fv_hbm, ov_hbm, kv_hbm, cv_hbm, fr_hbm, orr_hbm, kr_hbm,
               cr_hbm,
               st_s, st_d, bs0, bo0, bs1, bo1, fl2, kst, hist):
        c = lax.axis_index("c")
        s = lax.axis_index("s")
        iota = lax.iota(I32, 16)
        ones = jnp.ones((16,), F32)
        zeros = jnp.zeros((16,), F32)

        def bin_et(src_hbm, dst_hbm, f_out, o_out, k_out, c_out):
            bufs = ((bs0, bo0), (bs1, bo1))

            def flush_chunk(bkt, off_in_buf, k):
                bsrc, boff = bufs[bkt]
                for r in range(8):
                    for j in range(8):
                        fl2[r, pl.ds(j * 16, 16)] = (
                            boff[pl.ds(off_in_buf + r * _EB + j * 16, 16)])
                pltpu.sync_copy(
                    bsrc.at[pl.ds(off_in_buf, _CHE)],
                    f_out.at[bkt, s, 0, pl.ds(k * _CHE, _CHE)])
                pltpu.sync_copy(fl2, o_out.at[bkt, s, pl.ds(k * 8, 8)])

            def maybe_flush(bkt):
                def do(args):
                    f, k = args
                    flush_chunk(bkt, 0, k)
                    bsrc, boff = bufs[bkt]
                    for j in range(9):
                        t = bsrc[pl.ds(_CHE + j * 16, 16)]
                        bsrc[pl.ds(j * 16, 16)] = t
                        t2 = boff[pl.ds(_CHE + j * 16, 16)]
                        boff[pl.ds(j * 16, 16)] = t2
                    return (f - _CHE, k + 1)

                def keep(args):
                    return args

                return lambda f, k: lax.cond(f >= _CHE, do, keep, (f, k))

            def chunk_body(ch, carry):
                row0 = s * per_tile + ch * 32
                pltpu.sync_copy(src_hbm.at[pl.ds(row0, 32)], st_s)
                pltpu.sync_copy(dst_hbm.at[pl.ds(row0, 32)], st_d)

                def row_body(r, carry):
                    f0, k0, f1, k1 = carry
                    for j in range(8):
                        sv = st_s[r, pl.ds(j * 16, 16)]
                        dv = st_d[r, pl.ds(j * 16, 16)]
                        plsc.addupdate_scatter(hist, [dv], ones)
                        m0 = dv < half
                        n0 = jnp.sum(m0.astype(I32), axis=0)
                        plsc.store_compressed(bs0.at[pl.ds(f0, 16)], sv, mask=m0)
                        plsc.store_compressed(bo0.at[pl.ds(f0, 16)], dv, mask=m0)
                        m1 = jnp.logical_not(m0)
                        plsc.store_compressed(bs1.at[pl.ds(f1, 16)], sv, mask=m1)
                        plsc.store_compressed(
                            bo1.at[pl.ds(f1, 16)], dv - half, mask=m1)
                        f0 = f0 + n0
                        f1 = f1 + (16 - n0)
                    f0, k0 = maybe_flush(0)(f0, k0)
                    f1, k1 = maybe_flush(1)(f1, k1)
                    return (f0, k0, f1, k1)

                return lax.fori_loop(0, 32, row_body, carry)

            @pl.loop(0, hs // 16)
            def _(i):
                hist[pl.ds(i * 16, 16)] = zeros

            f0, k0, f1, k1 = lax.fori_loop(
                0, per_tile // 32, chunk_body, (0, 0, 0, 0))
            pltpu.sync_copy(hist, c_out.at[s, 0])

            def drain(bkt, f, k):
                bsrc, boff = bufs[bkt]
                bsrc[pl.ds(f, 16)] = iota + ((f * 37) & 16383)
                boff[pl.ds(f, 16)] = half + ((iota + f) & 63)
                f = (f & ~15) + 16

                def wcond(st):
                    return (st[0] & (_CHE - 1)) != 0

                def wbody(st):
                    fw = st[0]
                    bsrc[pl.ds(fw, 16)] = iota + ((fw * 37) & 16383)
                    boff[pl.ds(fw, 16)] = half + ((iota + fw) & 63)
                    return (fw + 16,)

                f = lax.while_loop(wcond, wbody, (f,))[0]

                def fl(i, kk):
                    flush_chunk(bkt, i * _CHE, kk)
                    return kk + 1

                return lax.fori_loop(0, f // _CHE, fl, k)

            k0 = drain(0, f0, k0)
            k1 = drain(1, f1, k1)
            kst[...] = jnp.broadcast_to(k0, (16,)).astype(I32)
            pltpu.sync_copy(kst, k_out.at[0, s, 0])
            kst[...] = jnp.broadcast_to(k1, (16,)).astype(I32)
            pltpu.sync_copy(kst, k_out.at[1, s, 0])

        @pl.when(c == 0)
        def _():
            bin_et(sv_hbm, dv_hbm, fv_hbm, ov_hbm, kv_hbm, cv_hbm)

        @pl.when(c == 1)
        def _():
            bin_et(sr_hbm, dr_hbm, fr_hbm, orr_hbm, kr_hbm, cr_hbm)

    return binker(sv2d, dv2d, sr2d, dr2d)


@functools.partial(jax.jit, static_argnames=("n_dst",))
def _segsum_b(fsrc, foff, kcnt, t0, t1, zz, *, n_dst):
    half = n_dst // 2
    stripe = (half // _NSUB) // 8 * 8
    rem = half - _NSUB * stripe
    acc_rows = _ACC_ROWS
    assert acc_rows >= half + 64

    @functools.partial(
        pl.kernel,
        out_type=(jax.ShapeDtypeStruct((n_dst, 64), F32),
                  jax.ShapeDtypeStruct((n_dst, 64), F32)),
        mesh=_mesh(),
        compiler_params=_sc_params(tc_tiling=False),
        scratch_types=[
            pltpu.VMEM((_CHE,), I32),
            pltpu.VMEM((8, _EB), I32),
            pltpu.VMEM((_EB, 64), F32),
            pltpu.VMEM((_EB, 64), F32),
            pltpu.VMEM((16,), I32),
            pltpu.VMEM_SHARED((acc_rows, 64), F32),
            pltpu.SemaphoreType.DMA,
            pltpu.SemaphoreType.DMA,
            pltpu.SemaphoreType.DMA,
            pltpu.SemaphoreType.DMA,
        ],
    )
    def seg(fsrc_hbm, foff_hbm, kcnt_hbm, t0_hbm, t1_hbm, zz_hbm,
            s0_hbm, s1_hbm,
            src_st, off_st, rows0, rows1, kst, acc, gs0, gs1, ss0, ss1):
        c = lax.axis_index("c")
        s = lax.axis_index("s")
        base = c * half
        zstripe = acc_rows // _NSUB
        rows = (rows0, rows1)
        gsem = (gs0, gs1)
        ssem = (ss0, ss1)

        pltpu.sync_copy(kcnt_hbm.at[c, s, 0], kst)
        nck = jnp.max(kst[...], axis=0)

        for t_hbm, s_hbm in ((t0_hbm, s0_hbm), (t1_hbm, s1_hbm)):
            pltpu.sync_copy(zz_hbm, acc.at[pl.ds(s * zstripe, zstripe)])
            plsc.subcore_barrier()

            @pl.loop(0, nck)
            def _(k):
                pltpu.sync_copy(
                    fsrc_hbm.at[c, s, 0, pl.ds(k * _CHE, _CHE)], src_st)
                pltpu.sync_copy(foff_hbm.at[c, s, pl.ds(k * 8, 8)], off_st)

                gd = [None] * 8
                sd = [None] * 8
                gd[0] = pltpu.async_copy(
                    t_hbm.at[src_st.at[pl.ds(0, _EB)]], rows[0], gsem[0])
                for b in range(8):
                    i = b & 1
                    if b + 1 < 8:
                        if b >= 1:
                            sd[b - 1].wait()
                        gd[b + 1] = pltpu.async_copy(
                            t_hbm.at[src_st.at[pl.ds((b + 1) * _EB, _EB)]],
                            rows[1 - i], gsem[1 - i])
                    gd[b].wait()
                    sd[b] = pltpu.async_copy(
                        rows[i], acc.at[off_st.at[b]], ssem[i], add=True)
                sd[6].wait()
                sd[7].wait()

            plsc.subcore_barrier()
            pltpu.sync_copy(acc.at[pl.ds(s * stripe, stripe)],
                            s_hbm.at[pl.ds(base + s * stripe, stripe)])

            @pl.when(s == _NSUB - 1)
            def _():
                pltpu.sync_copy(acc.at[pl.ds(_NSUB * stripe, rem)],
                                s_hbm.at[pl.ds(base + _NSUB * stripe, rem)])

            plsc.subcore_barrier()

    return seg(fsrc, foff, kcnt, t0, t1, zz)


@functools.partial(jax.jit, static_argnames=("hs",))
def _counts(dv2d, dr2d, *, hs):
    rows2d = dv2d.shape[0]
    per_w = rows2d // (_NSUB * _NCORE)

    @functools.partial(
        pl.kernel,
        out_type=(jax.ShapeDtypeStruct((_NSUB * _NCORE, 1, hs), F32),
                  jax.ShapeDtypeStruct((_NSUB * _NCORE, 1, hs), F32)),
        mesh=_mesh(),
        compiler_params=_sc_params(),
        scratch_types=[
            pltpu.VMEM((per_w, _EB), I32),
            pltpu.VMEM((hs,), F32),
        ],
    )
    def cnts(dv_hbm, dr_hbm, cv_hbm, cr_hbm, dst_st, hist):
        c = lax.axis_index("c")
        s = lax.axis_index("s")
        wid = s * _NCORE + c
        ones = jnp.ones((16,), F32)
        zeros = jnp.zeros((16,), F32)
        for d_hbm, o_hbm in ((dv_hbm, cv_hbm), (dr_hbm, cr_hbm)):
            @pl.loop(0, hs // 16)
            def _(i):
                hist[pl.ds(i * 16, 16)] = zeros

            pltpu.sync_copy(d_hbm.at[pl.ds(wid * per_w, per_w)], dst_st)

            @pl.loop(0, per_w)
            def _(b):
                for j in range(_EB // 16):
                    d = dst_st[b, pl.ds(j * 16, 16)]
                    plsc.addupdate_scatter(hist, [d], ones)

            pltpu.sync_copy(hist, o_hbm.at[wid, 0])

    return cnts(dv2d, dr2d)


def _combine_body(s0_ref, s1_ref, c_ref, x0_ref, x1_ref, wl_ref, wr_ref,
                  b_ref, *outs, relu, split):
    cnt = jnp.sum(c_ref[...], axis=1)
    cl = jnp.maximum(cnt, 1.0)[:, None]
    wl = wl_ref[...]
    wr = wr_ref[...]
    acc = (jnp.dot(s0_ref[...] / cl, wl[:64], preferred_element_type=F32)
           + jnp.dot(s1_ref[...] / cl, wl[64:], preferred_element_type=F32)
           + jnp.dot(x0_ref[...], wr[:64], preferred_element_type=F32)
           + jnp.dot(x1_ref[...], wr[64:], preferred_element_type=F32)
           + b_ref[...])
    if relu:
        acc = jnp.maximum(acc, 0.0)
    if split:
        outs[0][...] = acc[:, :64]
        outs[1][...] = acc[:, 64:]
    else:
        outs[0][...] = acc


@functools.partial(jax.jit, static_argnames=("relu", "split"))
def _combine(s0, s1, cnt, x0, x1, wl, wr, b, *, relu, split):
    n = s0.shape[0]
    r = 2000
    grid = n // r
    nw = cnt.shape[1]
    if split:
        out_shape = (jax.ShapeDtypeStruct((n, 64), F32),
                     jax.ShapeDtypeStruct((n, 64), F32))
        out_specs = (pl.BlockSpec((r, 64), lambda i: (i, 0)),
                     pl.BlockSpec((r, 64), lambda i: (i, 0)))
    else:
        out_shape = jax.ShapeDtypeStruct((n, 128), F32)
        out_specs = pl.BlockSpec((r, 128), lambda i: (i, 0))
    return pl.pallas_call(
        functools.partial(_combine_body, relu=relu, split=split),
        grid=(grid,),
        in_specs=[
            pl.BlockSpec((r, 64), lambda i: (i, 0)),
            pl.BlockSpec((r, 64), lambda i: (i, 0)),
            pl.BlockSpec((r, nw), lambda i: (i, 0)),
            pl.BlockSpec((r, 64), lambda i: (i, 0)),
            pl.BlockSpec((r, 64), lambda i: (i, 0)),
            pl.BlockSpec((128, 128), lambda i: (0, 0)),
            pl.BlockSpec((128, 128), lambda i: (0, 0)),
            pl.BlockSpec((1, 128), lambda i: (0, 0)),
        ],
        out_specs=out_specs,
        out_shape=out_shape,
    )(s0, s1, cnt, x0, x1, wl, wr, b.reshape(1, 128))


def _prep_edges(ei, n_dst):
    src = ei[0].astype(I32)
    dst = ei[1].astype(I32)
    e = src.shape[0]
    ep = -(-e // 32768) * 32768
    fill = jnp.arange(ep - e, dtype=I32)
    psrc = fill % n_dst
    pdst = n_dst + (fill % 64)
    if e % _NSUB == 0 and (ep - e) % _NSUB == 0:
        src = jnp.concatenate(
            [src.reshape(_NSUB, -1), psrc.reshape(_NSUB, -1)], axis=1)
        dst = jnp.concatenate(
            [dst.reshape(_NSUB, -1), pdst.reshape(_NSUB, -1)], axis=1)
    else:
        src = jnp.concatenate([src, psrc])
        dst = jnp.concatenate([dst, pdst])
    return src.reshape(-1, _EB), dst.reshape(-1, _EB)


def kernel(x_person, x_product, edge_index_viewed, edge_index_rev,
           W_l0_v, b0_v, W_r0_v, W_l0_r, b0_r, W_r0_r,
           W_l1_v, b1_v, W_r1_v, W_l1_r, b1_r, W_r1_r):
    np_, nq = x_person.shape[0], x_product.shape[0]
    xp0, xp1 = x_person[:, :64], x_person[:, 64:]
    xq0, xq1 = x_product[:, :64], x_product[:, 64:]
    srcv, dstv = _prep_edges(edge_index_viewed, nq)
    srcr, dstr = _prep_edges(edge_index_rev, np_)
    zz = jnp.zeros((_ACC_ROWS // _NSUB, 64), F32)
    hs = -(-(max(np_, nq) + 64) // 16) * 16

    fv, ov, kv, cvp, fr, orr, kr, crp = _binedges(
        srcv, dstv, srcr, dstr, n_dst=nq, hs=hs)
    cv = cvp[:, 0, :nq].T
    cr = crp[:, 0, :np_].T

    sv0, sv1 = _segsum_b(fv, ov, kv, xp0, xp1, zz, n_dst=nq)
    sr0, sr1 = _segsum_b(fr, orr, kr, xq0, xq1, zz, n_dst=np_)
    hp0, hp1 = _combine(sr0, sr1, cr, xp0, xp1, W_l0_r, W_r0_r, b0_r,
                        relu=True, split=True)
    hq0, hq1 = _combine(sv0, sv1, cv, xq0, xq1, W_l0_v, W_r0_v, b0_v,
                        relu=True, split=True)

    sv0b, sv1b = _segsum_b(fv, ov, kv, hp0, hp1, zz, n_dst=nq)
    sr0b, sr1b = _segsum_b(fr, orr, kr, hq0, hq1, zz, n_dst=np_)
    h_prod2 = _combine(sv0b, sv1b, cv, hq0, hq1, W_l1_v, W_r1_v, b1_v,
                       relu=False, split=False)
    h_pers2 = _combine(sr0b, sr1b, cr, hp0, hp1, W_l1_r, W_r1_r, b1_r,
                       relu=False, split=False)
    return (h_pers2, h_prod2)
